# baseline (speedup 1.0000x reference)
import functools

import jax
import jax.numpy as jnp
from jax import lax
from jax.experimental import pallas as pl
from jax.experimental.pallas import tpu as pltpu
from jax.experimental.pallas import tpu_sc as plsc

_ROWS = 128
_COLS = 32768
_NC = 2
_NS = 16
_NW = _NC * _NS
_RPW = _ROWS // _NW
_CHUNK = 8192
_NBUF = 4
_NCHUNK = _COLS // _CHUNK
_NGLOBAL = _RPW * _NCHUNK

_mesh = plsc.VectorSubcoreMesh(core_axis_name="c", subcore_axis_name="s")


@functools.partial(
    pl.kernel,
    mesh=_mesh,
    out_type=jax.ShapeDtypeStruct((_NW * 16,), jnp.int32),
    scratch_types=[
        pltpu.VMEM((_NBUF, _CHUNK), jnp.float32),
        pltpu.VMEM((16,), jnp.int32),
        pltpu.SemaphoreType.DMA,
        pltpu.SemaphoreType.DMA,
        pltpu.SemaphoreType.DMA,
        pltpu.SemaphoreType.DMA,
        pltpu.SemaphoreType.DMA,
    ],
    compiler_params=pltpu.CompilerParams(needs_layout_passes=False),
)
def _sc_counts(scores_hbm, counts_hbm, inbuf, cvec, isem0, isem1, isem2,
               isem3, csem):
    isems = (isem0, isem1, isem2, isem3)
    cid = lax.axis_index("c")
    sid = lax.axis_index("s")
    wid = sid * _NC + cid
    base = wid * _RPW

    def in_copy(g, buf):
        row = base + g // _NCHUNK
        off = (g % _NCHUNK) * _CHUNK
        return pltpu.make_async_copy(
            scores_hbm.at[row, pl.ds(off, _CHUNK)],
            inbuf.at[buf],
            isems[buf],
        )

    for g in range(_NBUF - 1):
        in_copy(g, g % _NBUF).start()

    def count_chunk(buf, pivot, acc):
        def body(i, a):
            v = inbuf[buf, pl.ds(i * 16, 16)]
            return a + plsc.all_reduce_population_count(v > pivot)
        return lax.fori_loop(0, _CHUNK // 16, body, acc, unroll=16)

    lanes = lax.iota(jnp.int32, 16)
    count_vec = jnp.zeros((16,), jnp.int32)
    for r in range(_RPW):
        acc = jnp.zeros((16,), jnp.int32)
        pivot = jnp.float32(0)
        for ch in range(_NCHUNK):
            g = r * _NCHUNK + ch
            if g + _NBUF - 1 < _NGLOBAL:
                in_copy(g + _NBUF - 1, (g + _NBUF - 1) % _NBUF).start()
            in_copy(g, g % _NBUF).wait()
            if ch == 0:
                pivot = inbuf[g % _NBUF, pl.ds(0, 16)][0]
            acc = count_chunk(g % _NBUF, pivot, acc)
        count_vec = jnp.where(lanes == r, acc[0], count_vec)

    cvec[...] = count_vec
    ccp = pltpu.make_async_copy(cvec, counts_hbm.at[pl.ds(wid * 16, 16)],
                                csem)
    ccp.start()
    ccp.wait()


_R_BLK = 16


def _expand_kernel(c_ref, o_ref):
    cnt = c_ref[:, :]
    iota = jax.lax.broadcasted_iota(jnp.int32, (_R_BLK, _COLS), 1)
    o_ref[:, :] = (iota == cnt).astype(jnp.float32)


def kernel(scores):
    counts = _sc_counts(scores)
    counts = counts.reshape(_NW, 16)[:, :_RPW].reshape(_ROWS, 1)
    return pl.pallas_call(
        _expand_kernel,
        out_shape=jax.ShapeDtypeStruct((_ROWS, _COLS), jnp.float32),
        grid=(_ROWS // _R_BLK,),
        in_specs=[pl.BlockSpec((_R_BLK, 1), lambda i: (i, 0))],
        out_specs=pl.BlockSpec((_R_BLK, _COLS), lambda i: (i, 0)),
    )(counts)
